# baseline (speedup 1.0000x reference)
import functools

import jax
import jax.numpy as jnp
from jax import lax
from jax.experimental import pallas as pl
from jax.experimental.pallas import tpu as pltpu

HID = 2048
MEM = 50000
TOPK = 64
BM = 400
NB = MEM // BM
BQ = 256


def _q_body(query_ref, wq_ref, bq_ref, q_ref):
    acc = lax.dot_general(
        query_ref[...], wq_ref[...],
        ((( 1,), (1,)), ((), ())),
        preferred_element_type=jnp.float32,
    )
    q_ref[...] = acc + bq_ref[...]


def _main_body(q_ref, keys_ref, imp_ref, age_ref, mv_ref,
               recalled_ref, values_ref,
               l_ref, mx_ref, rows_ref, vals_ref, sem):
    i = pl.program_id(0)
    logits = lax.dot_general(
        q_ref[...], keys_ref[...],
        (((1,), (1,)), ((), ())),
        preferred_element_type=jnp.float32,
    )
    imp = imp_ref[...].reshape(1, BM)
    age = age_ref[...].reshape(1, BM)
    logits = logits * imp / (1.0 + age)
    l_ref[pl.ds(i, 1), :] = logits
    mx_ref[pl.ds(i, 1), :] = jnp.max(logits, axis=1, keepdims=True)

    @pl.when(i == NB - 1)
    def _final():
        lfull = l_ref[...]
        gmax = jnp.max(mx_ref[...])
        denom = jnp.sum(jnp.exp(lfull - gmax))

        rio = lax.broadcasted_iota(jnp.int32, (NB, 1), 0)
        cio = lax.broadcasted_iota(jnp.int32, (1, BM), 1)
        k64 = lax.broadcasted_iota(jnp.int32, (1, TOPK), 1)

        def pick(j, _):
            mx = mx_ref[...]
            gm = jnp.max(mx)
            ridx = jnp.min(jnp.where(mx == gm, rio, NB))
            row = l_ref[pl.ds(ridx, 1), :]
            cidx = jnp.min(jnp.where(row == gm, cio, BM))
            flat = ridx * BM + cidx
            vals_ref[...] = jnp.where(k64 == j, gm, vals_ref[...])
            cp = pltpu.make_async_copy(
                mv_ref.at[pl.ds(flat, 1), :],
                rows_ref.at[pl.ds(j, 1), :], sem)
            cp.start()
            newrow = jnp.where(cio == cidx, -jnp.inf, row)
            l_ref[pl.ds(ridx, 1), :] = newrow
            mx_ref[pl.ds(ridx, 1), :] = jnp.max(newrow, axis=1,
                                                keepdims=True)
            return 0

        lax.fori_loop(0, TOPK, pick, 0)

        def drain(j, _):
            pltpu.make_async_copy(
                mv_ref.at[pl.ds(0, 1), :],
                rows_ref.at[pl.ds(0, 1), :], sem).wait()
            return 0

        lax.fori_loop(0, TOPK, drain, 0)

        w = jnp.exp(vals_ref[...] - gmax) / denom
        values_ref[...] = w
        recalled_ref[...] = lax.dot_general(
            w, rows_ref[...],
            (((1,), (0,)), ((), ())),
            preferred_element_type=jnp.float32,
        )


def kernel(query, Wq, bq, memory_keys, memory_values, memory_importance,
           memory_age, top_k):
    del top_k
    query2 = query.reshape(1, HID)
    bq2 = bq.reshape(1, HID)
    imp3 = memory_importance.reshape(NB, 1, BM)
    age3 = memory_age.reshape(NB, 1, BM)

    q = pl.pallas_call(
        _q_body,
        grid=(HID // BQ,),
        in_specs=[
            pl.BlockSpec((1, HID), lambda i: (0, 0)),
            pl.BlockSpec((BQ, HID), lambda i: (i, 0)),
            pl.BlockSpec((1, BQ), lambda i: (0, i)),
        ],
        out_specs=pl.BlockSpec((1, BQ), lambda i: (0, i)),
        out_shape=jax.ShapeDtypeStruct((1, HID), jnp.float32),
    )(query2, Wq, bq2)

    recalled, values = pl.pallas_call(
        _main_body,
        grid=(NB,),
        in_specs=[
            pl.BlockSpec((1, HID), lambda i: (0, 0)),
            pl.BlockSpec((BM, HID), lambda i: (i, 0)),
            pl.BlockSpec((1, 1, BM), lambda i: (i, 0, 0)),
            pl.BlockSpec((1, 1, BM), lambda i: (i, 0, 0)),
            pl.BlockSpec(memory_space=pltpu.MemorySpace.HBM),
        ],
        out_specs=[
            pl.BlockSpec((1, HID), lambda i: (0, 0)),
            pl.BlockSpec((1, TOPK), lambda i: (0, 0)),
        ],
        out_shape=[
            jax.ShapeDtypeStruct((1, HID), jnp.float32),
            jax.ShapeDtypeStruct((1, TOPK), jnp.float32),
        ],
        scratch_shapes=[
            pltpu.VMEM((NB, BM), jnp.float32),
            pltpu.VMEM((NB, 1), jnp.float32),
            pltpu.VMEM((TOPK, HID), jnp.float32),
            pltpu.VMEM((1, TOPK), jnp.float32),
            pltpu.SemaphoreType.DMA,
        ],
    )(q, memory_keys, imp3, age3, memory_values)

    return recalled.reshape(HID), values.reshape(TOPK)


# speedup vs baseline: 1.2655x; 1.2655x over previous
import jax
import jax.numpy as jnp
from jax import lax
from jax.experimental import pallas as pl
from jax.experimental.pallas import tpu as pltpu

HID = 2048
MEM = 50000
TOPK = 64
BM = 1000
NB = MEM // BM
NEG = float("-inf")


def _main_body(query_ref, wq_ref, bq_ref, keys_ref, imp_ref, age_ref,
               mv_ref, recalled_ref, values_ref,
               q_ref, l_ref, rows_ref, sem):
    i = pl.program_id(0)

    @pl.when(i == 0)
    def _project():
        q_ref[...] = lax.dot_general(
            query_ref[...], wq_ref[...],
            (((1,), (1,)), ((), ())),
            preferred_element_type=jnp.float32,
        ) + bq_ref[...]

    logits = lax.dot_general(
        q_ref[...], keys_ref[...],
        (((1,), (1,)), ((), ())),
        preferred_element_type=jnp.float32,
    )
    imp = imp_ref[...].reshape(1, BM)
    age = age_ref[...].reshape(1, BM)
    logits = logits * imp / (1.0 + age)
    l_ref[pl.ds(i, 1), :] = logits

    @pl.when(i == NB - 1)
    def _final():
        lfull = l_ref[...]
        mxcol = jnp.max(lfull, axis=1, keepdims=True)
        rio = lax.broadcasted_iota(jnp.int32, (NB, 1), 0)
        lio = lax.broadcasted_iota(jnp.int32, (1, 128), 1)
        cio = lax.broadcasted_iota(jnp.int32, (1, BM), 1)
        kio = lax.broadcasted_iota(jnp.int32, (1, TOPK), 1)
        eye = (rio == lio).astype(jnp.float32)
        mx0 = lax.dot_general(
            mxcol, eye, (((0,), (0,)), ((), ())),
            preferred_element_type=jnp.float32,
        )
        mx0 = jnp.where(lio < NB, mx0, NEG)
        gmax = jnp.max(mx0)
        denom = jnp.sum(jnp.exp(lfull - gmax))

        def pick(j, carry):
            mx, vals = carry
            gm = jnp.max(mx)
            ridx = jnp.min(jnp.where(mx == gm, lio, NB))
            row = l_ref[pl.ds(ridx, 1), :]
            cidx = jnp.min(jnp.where(row == gm, cio, BM))
            flat = ridx * BM + cidx
            pltpu.make_async_copy(
                mv_ref.at[pl.ds(flat, 1), :],
                rows_ref.at[pl.ds(j, 1), :], sem).start()
            newrow = jnp.where(cio == cidx, NEG, row)
            l_ref[pl.ds(ridx, 1), :] = newrow
            mx = jnp.where(lio == ridx, jnp.max(newrow), mx)
            vals = jnp.where(kio == j, gm, vals)
            return mx, vals

        _, vals = lax.fori_loop(
            0, TOPK, pick,
            (mx0, jnp.full((1, TOPK), NEG, jnp.float32)))

        def drain(j, _):
            pltpu.make_async_copy(
                mv_ref.at[pl.ds(0, 1), :],
                rows_ref.at[pl.ds(0, 1), :], sem).wait()
            return 0

        lax.fori_loop(0, TOPK, drain, 0)

        w = jnp.exp(vals - gmax) / denom
        values_ref[...] = w
        recalled_ref[...] = lax.dot_general(
            w, rows_ref[...],
            (((1,), (0,)), ((), ())),
            preferred_element_type=jnp.float32,
        )


def kernel(query, Wq, bq, memory_keys, memory_values, memory_importance,
           memory_age, top_k):
    del top_k
    query2 = query.reshape(1, HID)
    bq2 = bq.reshape(1, HID)
    imp3 = memory_importance.reshape(NB, 1, BM)
    age3 = memory_age.reshape(NB, 1, BM)

    recalled, values = pl.pallas_call(
        _main_body,
        grid=(NB,),
        in_specs=[
            pl.BlockSpec((1, HID), lambda i: (0, 0)),
            pl.BlockSpec((HID, HID), lambda i: (0, 0)),
            pl.BlockSpec((1, HID), lambda i: (0, 0)),
            pl.BlockSpec((BM, HID), lambda i: (i, 0)),
            pl.BlockSpec((1, 1, BM), lambda i: (i, 0, 0)),
            pl.BlockSpec((1, 1, BM), lambda i: (i, 0, 0)),
            pl.BlockSpec(memory_space=pltpu.MemorySpace.HBM),
        ],
        out_specs=[
            pl.BlockSpec((1, HID), lambda i: (0, 0)),
            pl.BlockSpec((1, TOPK), lambda i: (0, 0)),
        ],
        out_shape=[
            jax.ShapeDtypeStruct((1, HID), jnp.float32),
            jax.ShapeDtypeStruct((1, TOPK), jnp.float32),
        ],
        scratch_shapes=[
            pltpu.VMEM((1, HID), jnp.float32),
            pltpu.VMEM((NB, BM), jnp.float32),
            pltpu.VMEM((TOPK, HID), jnp.float32),
            pltpu.SemaphoreType.DMA,
        ],
    )(query2, Wq, bq2, memory_keys, imp3, age3, memory_values)

    return recalled.reshape(HID), values.reshape(TOPK)
